# baseline (speedup 1.0000x reference)
import jax
import jax.numpy as jnp
from jax.experimental import pallas as pl
from jax.experimental.pallas import tpu as pltpu

LAT = 128


def _rows_block(n):
    for b in (2048, 2000, 1024, 1000, 512, 500, 256, 250, 128, 125, 64, 32, 16, 8):
        if n % b == 0:
            return b
    return n


def _ln(h, g, bb):
    mu = jnp.mean(h, axis=-1, keepdims=True)
    var = jnp.mean((h - mu) * (h - mu), axis=-1, keepdims=True)
    return (h - mu) / jnp.sqrt(var + 1e-5) * g + bb


def _dot(a, b):
    return jnp.dot(a, b, preferred_element_type=jnp.float32)


def _enc_body(x_ref, w1_ref, b1_ref, w2_ref, b2_ref, w3_ref, b3_ref,
              g_ref, bb_ref, o_ref):
    x = x_ref[...]
    h = jnp.maximum(_dot(x, w1_ref[...]) + b1_ref[...], 0.0)
    h = jnp.maximum(_dot(h, w2_ref[...]) + b2_ref[...], 0.0)
    h = _dot(h, w3_ref[...]) + b3_ref[...]
    o_ref[...] = _ln(h, g_ref[...], bb_ref[...])


def _dec_body(x_ref, w1_ref, b1_ref, w2_ref, b2_ref, w3_ref, b3_ref, o_ref):
    x = x_ref[...]
    h = jnp.maximum(_dot(x, w1_ref[...]) + b1_ref[...], 0.0)
    h = jnp.maximum(_dot(h, w2_ref[...]) + b2_ref[...], 0.0)
    o_ref[...] = _dot(h, w3_ref[...]) + b3_ref[...]


def _edge_body(e_ref, gs_ref, gd_ref, w1_ref, b1_ref, w2_ref, b2_ref,
               w3_ref, b3_ref, lng_ref, lnb_ref, ne_ref, enew_ref):
    e = e_ref[...]
    x = jnp.concatenate([e, gs_ref[...], gd_ref[...]], axis=-1)
    h = jnp.maximum(_dot(x, w1_ref[...]) + b1_ref[...], 0.0)
    h = jnp.maximum(_dot(h, w2_ref[...]) + b2_ref[...], 0.0)
    h = _dot(h, w3_ref[...]) + b3_ref[...]
    ne = _ln(h, lng_ref[...], lnb_ref[...])
    ne_ref[...] = ne
    enew_ref[...] = e + ne


def _node_body(n_ref, a_ref, w1_ref, b1_ref, w2_ref, b2_ref,
               w3_ref, b3_ref, lng_ref, lnb_ref, o_ref):
    n = n_ref[...]
    x = jnp.concatenate([n, a_ref[...]], axis=-1)
    h = jnp.maximum(_dot(x, w1_ref[...]) + b1_ref[...], 0.0)
    h = jnp.maximum(_dot(h, w2_ref[...]) + b2_ref[...], 0.0)
    h = _dot(h, w3_ref[...]) + b3_ref[...]
    o_ref[...] = n + _ln(h, lng_ref[...], lnb_ref[...])


def _mk_call(body, nrows, row_in_widths, weight_shapes, row_out_widths):
    B = _rows_block(nrows)
    grid = (nrows // B,)
    in_specs = [pl.BlockSpec((B, w), lambda i: (i, 0)) for w in row_in_widths]
    for shp in weight_shapes:
        in_specs.append(pl.BlockSpec(shp, lambda i: (0,) * len(shp)))
    out_specs = [pl.BlockSpec((B, w), lambda i: (i, 0)) for w in row_out_widths]
    out_shape = [jax.ShapeDtypeStruct((nrows, w), jnp.float32) for w in row_out_widths]
    if len(row_out_widths) == 1:
        out_specs = out_specs[0]
        out_shape = out_shape[0]
    return pl.pallas_call(
        body,
        grid=grid,
        in_specs=in_specs,
        out_specs=out_specs,
        out_shape=out_shape,
    )


def _wln(p):
    return [p["w"][0], p["b"][0][None, :], p["w"][1], p["b"][1][None, :],
            p["w"][2], p["b"][2][None, :], p["g"][None, :], p["bb"][None, :]]


def _enc(p, x):
    nrows, din = x.shape
    ws = _wln(p)
    call = _mk_call(_enc_body, nrows, [din], [w.shape for w in ws], [LAT])
    return call(x, *ws)


def _dec(p, x, dout):
    nrows = x.shape[0]
    w3 = jnp.zeros((LAT, LAT), jnp.float32).at[:, :dout].set(p["w"][2])
    b3 = jnp.zeros((1, LAT), jnp.float32).at[0, :dout].set(p["b"][2])
    ws = [p["w"][0], p["b"][0][None, :], p["w"][1], p["b"][1][None, :], w3, b3]
    call = _mk_call(_dec_body, nrows, [LAT], [w.shape for w in ws], [LAT])
    return call(x, *ws)[:, :dout]


def _edge_mlp(p, e, gs, gd):
    nrows = e.shape[0]
    ws = _wln(p)
    call = _mk_call(_edge_body, nrows, [LAT, LAT, LAT],
                    [w.shape for w in ws], [LAT, LAT])
    return call(e, gs, gd, *ws)


def _node_mlp(p, n, a):
    nrows = n.shape[0]
    ws = _wln(p)
    call = _mk_call(_node_body, nrows, [LAT, LAT], [w.shape for w in ws], [LAT])
    return call(n, a, *ws)


def _fold_norm(mlp, nrm):
    std = jnp.maximum(nrm["std"], 1e-8)
    w1 = mlp["w"][0] / std[:, None]
    b1 = mlp["b"][0] - (nrm["mean"] / std) @ mlp["w"][0]
    out = dict(mlp)
    out["w"] = [w1] + list(mlp["w"][1:])
    out["b"] = [b1] + list(mlp["b"][1:])
    return out


def kernel(mesh_x, obj_x, mo_edge_attr, om_edge_attr, mo_edge_index,
           om_edge_index, params):
    P = params
    NMr = mesh_x.shape[0]
    NOr = obj_x.shape[0]

    m = _enc(_fold_norm(P["enc_mesh"], P["node_norm"]), mesh_x)
    o = _enc(_fold_norm(P["enc_obj"], P["node_norm"]), obj_x)
    e_mo = _enc(_fold_norm(P["enc_mo"], P["mo_norm"]), mo_edge_attr)
    e_om = _enc(_fold_norm(P["enc_om"], P["om_norm"]), om_edge_attr)

    mo_src, mo_dst = mo_edge_index[0], mo_edge_index[1]
    om_src, om_dst = om_edge_index[0], om_edge_index[1]

    for s in range(len(P["proc"])):
        ps = P["proc"][s]
        ne_mo, e_mo = _edge_mlp(ps["e_mo"], e_mo,
                                jnp.take(m, mo_src, axis=0),
                                jnp.take(o, mo_dst, axis=0))
        ne_om, e_om = _edge_mlp(ps["e_om"], e_om,
                                jnp.take(o, om_src, axis=0),
                                jnp.take(m, om_dst, axis=0))
        agg_m = jax.ops.segment_sum(ne_om, om_dst, num_segments=NMr)
        agg_o = jax.ops.segment_sum(ne_mo, mo_dst, num_segments=NOr)
        m = _node_mlp(ps["n_mesh"], m, agg_m)
        o = _node_mlp(ps["n_obj"], o, agg_o)

    m_pred = _dec(P["dec_mesh"], m, 3)
    o_pred = _dec(P["dec_obj"], o, 3)
    return m_pred, o_pred


# speedup vs baseline: 1.0092x; 1.0092x over previous
import jax
import jax.numpy as jnp
from jax import lax
from jax.experimental import pallas as pl
from jax.experimental.pallas import tpu as pltpu

LAT = 128
DN = (((1,), (0,)), ((), ()))


def _rows_block(n):
    for b in (2000, 1024, 1000, 512, 500, 256, 250, 128, 125, 64, 32, 16, 8):
        if n % b == 0:
            return b
    return n


def _mxdot(xb, w):
    return lax.dot_general(xb, w, DN, preferred_element_type=jnp.float32)


def _ln(h, g, bb):
    mu = jnp.sum(h, axis=-1, keepdims=True) * (1.0 / LAT)
    d = h - mu
    var = jnp.sum(d * d, axis=-1, keepdims=True) * (1.0 / LAT)
    return d / jnp.sqrt(var + 1e-5) * g + bb


def _bf(x):
    return x.astype(jnp.bfloat16)


def _tail2(h1b, w2, b2, w3, b3, g, bb):
    h2 = _bf(jnp.maximum(_mxdot(h1b, w2) + b2, 0.0))
    h3 = _mxdot(h2, w3) + b3
    return _ln(h3, g, bb)


def _enc_body(x_ref, w1_ref, b1_ref, w2_ref, b2_ref, w3_ref, b3_ref,
              g_ref, bb_ref, o_ref, ob_ref=None):
    x = x_ref[...]
    h1 = _bf(jnp.maximum(
        jnp.dot(x, w1_ref[...], preferred_element_type=jnp.float32)
        + b1_ref[...], 0.0))
    out = _tail2(h1, w2_ref[...], b2_ref[...], w3_ref[...], b3_ref[...],
                 g_ref[...], bb_ref[...])
    o_ref[...] = out
    if ob_ref is not None:
        ob_ref[...] = _bf(out)


def _dec_body(x_ref, w1_ref, b1_ref, w2_ref, b2_ref, w3_ref, b3_ref, o_ref):
    h1 = _bf(jnp.maximum(_mxdot(_bf(x_ref[...]), w1_ref[...]) + b1_ref[...], 0.0))
    h2 = _bf(jnp.maximum(_mxdot(h1, w2_ref[...]) + b2_ref[...], 0.0))
    o_ref[...] = _mxdot(h2, w3_ref[...]) + b3_ref[...]


def _edge_body(e_ref, gs_ref, gd_ref, w1_ref, b1_ref, w2_ref, b2_ref,
               w3_ref, b3_ref, lng_ref, lnb_ref, ne_ref, enew_ref):
    e = e_ref[...]
    xb = jnp.concatenate([_bf(e), gs_ref[...], gd_ref[...]], axis=-1)
    h1 = _bf(jnp.maximum(_mxdot(xb, w1_ref[...]) + b1_ref[...], 0.0))
    ne = _tail2(h1, w2_ref[...], b2_ref[...], w3_ref[...], b3_ref[...],
                lng_ref[...], lnb_ref[...])
    ne_ref[...] = ne
    enew_ref[...] = e + ne


def _node_body(n_ref, a_ref, w1_ref, b1_ref, w2_ref, b2_ref,
               w3_ref, b3_ref, lng_ref, lnb_ref, o_ref, ob_ref):
    n = n_ref[...]
    xb = jnp.concatenate([_bf(n), _bf(a_ref[...])], axis=-1)
    h1 = _bf(jnp.maximum(_mxdot(xb, w1_ref[...]) + b1_ref[...], 0.0))
    out = n + _tail2(h1, w2_ref[...], b2_ref[...], w3_ref[...], b3_ref[...],
                     lng_ref[...], lnb_ref[...])
    o_ref[...] = out
    ob_ref[...] = _bf(out)


def _mk_call(body, nrows, row_in, weight_shapes, row_out):
    B = _rows_block(nrows)
    grid = (nrows // B,)
    in_specs = [pl.BlockSpec((B, w), lambda i: (i, 0)) for w, _ in row_in]
    for shp in weight_shapes:
        in_specs.append(pl.BlockSpec(shp, lambda i: (0,) * len(shp)))
    out_specs = [pl.BlockSpec((B, w), lambda i: (i, 0)) for w, _ in row_out]
    out_shape = [jax.ShapeDtypeStruct((nrows, w), dt) for w, dt in row_out]
    if len(row_out) == 1:
        out_specs = out_specs[0]
        out_shape = out_shape[0]
    return pl.pallas_call(
        body,
        grid=grid,
        in_specs=in_specs,
        out_specs=out_specs,
        out_shape=out_shape,
    )


F32 = jnp.float32
BF16 = jnp.bfloat16


def _wln(p):
    return [p["w"][0], p["b"][0][None, :], p["w"][1], p["b"][1][None, :],
            p["w"][2], p["b"][2][None, :], p["g"][None, :], p["bb"][None, :]]


def _enc(p, x, with_bf16_table=False):
    nrows, din = x.shape
    ws = _wln(p)
    outs = [(LAT, F32)] + ([(LAT, BF16)] if with_bf16_table else [])
    call = _mk_call(_enc_body, nrows, [(din, F32)], [w.shape for w in ws], outs)
    return call(x, *ws)


def _dec(p, x, dout):
    nrows = x.shape[0]
    w3 = jnp.zeros((LAT, LAT), F32).at[:, :dout].set(p["w"][2])
    b3 = jnp.zeros((1, LAT), F32).at[0, :dout].set(p["b"][2])
    ws = [p["w"][0], p["b"][0][None, :], p["w"][1], p["b"][1][None, :], w3, b3]
    call = _mk_call(_dec_body, nrows, [(LAT, F32)], [w.shape for w in ws],
                    [(LAT, F32)])
    return call(x, *ws)[:, :dout]


def _edge_mlp(p, e, gs, gd):
    nrows = e.shape[0]
    ws = _wln(p)
    call = _mk_call(_edge_body, nrows, [(LAT, F32), (LAT, BF16), (LAT, BF16)],
                    [w.shape for w in ws], [(LAT, F32), (LAT, F32)])
    return call(e, gs, gd, *ws)


def _node_mlp(p, n, a):
    nrows = n.shape[0]
    ws = _wln(p)
    call = _mk_call(_node_body, nrows, [(LAT, F32), (LAT, F32)],
                    [w.shape for w in ws], [(LAT, F32), (LAT, BF16)])
    return call(n, a, *ws)


def _fold_norm(mlp, nrm):
    std = jnp.maximum(nrm["std"], 1e-8)
    w1 = mlp["w"][0] / std[:, None]
    b1 = mlp["b"][0] - (nrm["mean"] / std) @ mlp["w"][0]
    out = dict(mlp)
    out["w"] = [w1] + list(mlp["w"][1:])
    out["b"] = [b1] + list(mlp["b"][1:])
    return out


def kernel(mesh_x, obj_x, mo_edge_attr, om_edge_attr, mo_edge_index,
           om_edge_index, params):
    P = params
    NMr = mesh_x.shape[0]
    NOr = obj_x.shape[0]

    m, mb = _enc(_fold_norm(P["enc_mesh"], P["node_norm"]), mesh_x, True)
    o, ob = _enc(_fold_norm(P["enc_obj"], P["node_norm"]), obj_x, True)
    e_mo = _enc(_fold_norm(P["enc_mo"], P["mo_norm"]), mo_edge_attr)
    e_om = _enc(_fold_norm(P["enc_om"], P["om_norm"]), om_edge_attr)

    mo_src, mo_dst = mo_edge_index[0], mo_edge_index[1]
    om_src, om_dst = om_edge_index[0], om_edge_index[1]

    for s in range(len(P["proc"])):
        ps = P["proc"][s]
        ne_mo, e_mo = _edge_mlp(ps["e_mo"], e_mo,
                                jnp.take(mb, mo_src, axis=0),
                                jnp.take(ob, mo_dst, axis=0))
        ne_om, e_om = _edge_mlp(ps["e_om"], e_om,
                                jnp.take(ob, om_src, axis=0),
                                jnp.take(mb, om_dst, axis=0))
        agg_m = jax.ops.segment_sum(ne_om, om_dst, num_segments=NMr)
        agg_o = jax.ops.segment_sum(ne_mo, mo_dst, num_segments=NOr)
        m, mb = _node_mlp(ps["n_mesh"], m, agg_m)
        o, ob = _node_mlp(ps["n_obj"], o, agg_o)

    m_pred = _dec(P["dec_mesh"], m, 3)
    o_pred = _dec(P["dec_obj"], o, 3)
    return m_pred, o_pred


# speedup vs baseline: 1.6244x; 1.6095x over previous
import functools

import jax
import jax.numpy as jnp
from jax import lax
from jax.experimental import pallas as pl
from jax.experimental.pallas import tpu as pltpu
from jax.experimental.pallas import tpu_sc as plsc

LAT = 128
DN = (((1,), (0,)), ((), ()))

NW = 32
GBLK = 128


def _sc_gather4(m, o, mo_src, mo_dst, om_src, om_dst):
    E = mo_src.shape[0]
    nblk = E // GBLK
    per_tile = (nblk + NW - 1) // NW
    mesh = plsc.VectorSubcoreMesh(core_axis_name="c", subcore_axis_name="s")
    f32 = jnp.float32

    @functools.partial(
        pl.kernel, mesh=mesh,
        out_type=[jax.ShapeDtypeStruct((E, LAT), f32) for _ in range(4)],
        scratch_types=[
            pltpu.VMEM((GBLK,), jnp.int32),
            pltpu.VMEM((GBLK, LAT), f32),
            pltpu.SemaphoreType.DMA,
        ],
    )
    def k(m_hbm, o_hbm, ms_hbm, md_hbm, os_hbm, od_hbm,
          gs_mo, gd_mo, gs_om, gd_om, idxv, rowsv, sem):
        wid = lax.axis_index("s") * 2 + lax.axis_index("c")

        def do(tbl, idx_hbm, out_hbm, i):
            b = wid + NW * i

            @pl.when(b < nblk)
            def _():
                base = b * GBLK
                pltpu.sync_copy(idx_hbm.at[pl.ds(base, GBLK)], idxv)
                pltpu.async_copy(tbl.at[idxv], rowsv, sem).wait()
                pltpu.sync_copy(rowsv, out_hbm.at[pl.ds(base, GBLK)])

        def body(i, carry):
            do(m_hbm, ms_hbm, gs_mo, i)
            do(o_hbm, md_hbm, gd_mo, i)
            do(o_hbm, os_hbm, gs_om, i)
            do(m_hbm, od_hbm, gd_om, i)
            return carry

        lax.fori_loop(0, per_tile, body, 0)

    return k(m, o, mo_src, mo_dst, om_src, om_dst)


def _rows_block(n):
    for b in (2000, 1024, 1000, 512, 500, 256, 250, 128, 125, 64, 32, 16, 8):
        if n % b == 0:
            return b
    return n


def _mxdot(xb, w):
    return lax.dot_general(xb, w, DN, preferred_element_type=jnp.float32)


def _ln(h, g, bb):
    mu = jnp.sum(h, axis=-1, keepdims=True) * (1.0 / LAT)
    d = h - mu
    var = jnp.sum(d * d, axis=-1, keepdims=True) * (1.0 / LAT)
    return d / jnp.sqrt(var + 1e-5) * g + bb


def _bf(x):
    return x.astype(jnp.bfloat16)


def _tail2(h1b, w2, b2, w3, b3, g, bb):
    h2 = _bf(jnp.maximum(_mxdot(h1b, w2) + b2, 0.0))
    h3 = _mxdot(h2, w3) + b3
    return _ln(h3, g, bb)


def _enc_body(x_ref, w1_ref, b1_ref, w2_ref, b2_ref, w3_ref, b3_ref,
              g_ref, bb_ref, o_ref, ob_ref=None):
    x = x_ref[...]
    h1 = _bf(jnp.maximum(
        jnp.dot(x, w1_ref[...], preferred_element_type=jnp.float32)
        + b1_ref[...], 0.0))
    out = _tail2(h1, w2_ref[...], b2_ref[...], w3_ref[...], b3_ref[...],
                 g_ref[...], bb_ref[...])
    o_ref[...] = out
    if ob_ref is not None:
        ob_ref[...] = _bf(out)


def _dec_body(x_ref, w1_ref, b1_ref, w2_ref, b2_ref, w3_ref, b3_ref, o_ref):
    h1 = _bf(jnp.maximum(_mxdot(_bf(x_ref[...]), w1_ref[...]) + b1_ref[...], 0.0))
    h2 = _bf(jnp.maximum(_mxdot(h1, w2_ref[...]) + b2_ref[...], 0.0))
    o_ref[...] = _mxdot(h2, w3_ref[...]) + b3_ref[...]


def _edge_body(e_ref, gs_ref, gd_ref, w1_ref, b1_ref, w2_ref, b2_ref,
               w3_ref, b3_ref, lng_ref, lnb_ref, ne_ref, enew_ref):
    e = e_ref[...]
    xb = jnp.concatenate([_bf(e), _bf(gs_ref[...]), _bf(gd_ref[...])], axis=-1)
    h1 = _bf(jnp.maximum(_mxdot(xb, w1_ref[...]) + b1_ref[...], 0.0))
    ne = _tail2(h1, w2_ref[...], b2_ref[...], w3_ref[...], b3_ref[...],
                lng_ref[...], lnb_ref[...])
    ne_ref[...] = ne
    enew_ref[...] = e + ne


def _node_body(n_ref, a_ref, w1_ref, b1_ref, w2_ref, b2_ref,
               w3_ref, b3_ref, lng_ref, lnb_ref, o_ref):
    n = n_ref[...]
    xb = jnp.concatenate([_bf(n), _bf(a_ref[...])], axis=-1)
    h1 = _bf(jnp.maximum(_mxdot(xb, w1_ref[...]) + b1_ref[...], 0.0))
    o_ref[...] = n + _tail2(h1, w2_ref[...], b2_ref[...], w3_ref[...],
                            b3_ref[...], lng_ref[...], lnb_ref[...])


def _mk_call(body, nrows, row_in, weight_shapes, row_out):
    B = _rows_block(nrows)
    grid = (nrows // B,)
    in_specs = [pl.BlockSpec((B, w), lambda i: (i, 0)) for w, _ in row_in]
    for shp in weight_shapes:
        in_specs.append(pl.BlockSpec(shp, lambda i: (0,) * len(shp)))
    out_specs = [pl.BlockSpec((B, w), lambda i: (i, 0)) for w, _ in row_out]
    out_shape = [jax.ShapeDtypeStruct((nrows, w), dt) for w, dt in row_out]
    if len(row_out) == 1:
        out_specs = out_specs[0]
        out_shape = out_shape[0]
    return pl.pallas_call(
        body,
        grid=grid,
        in_specs=in_specs,
        out_specs=out_specs,
        out_shape=out_shape,
    )


F32 = jnp.float32
BF16 = jnp.bfloat16


def _wln(p):
    return [p["w"][0], p["b"][0][None, :], p["w"][1], p["b"][1][None, :],
            p["w"][2], p["b"][2][None, :], p["g"][None, :], p["bb"][None, :]]


def _enc(p, x):
    nrows, din = x.shape
    ws = _wln(p)
    call = _mk_call(_enc_body, nrows, [(din, F32)], [w.shape for w in ws],
                    [(LAT, F32)])
    return call(x, *ws)


def _dec(p, x, dout):
    nrows = x.shape[0]
    w3 = jnp.zeros((LAT, LAT), F32).at[:, :dout].set(p["w"][2])
    b3 = jnp.zeros((1, LAT), F32).at[0, :dout].set(p["b"][2])
    ws = [p["w"][0], p["b"][0][None, :], p["w"][1], p["b"][1][None, :], w3, b3]
    call = _mk_call(_dec_body, nrows, [(LAT, F32)], [w.shape for w in ws],
                    [(LAT, F32)])
    return call(x, *ws)[:, :dout]


def _edge_mlp(p, e, gs, gd):
    nrows = e.shape[0]
    ws = _wln(p)
    call = _mk_call(_edge_body, nrows, [(LAT, F32), (LAT, F32), (LAT, F32)],
                    [w.shape for w in ws], [(LAT, F32), (LAT, F32)])
    return call(e, gs, gd, *ws)


def _node_mlp(p, n, a):
    nrows = n.shape[0]
    ws = _wln(p)
    call = _mk_call(_node_body, nrows, [(LAT, F32), (LAT, F32)],
                    [w.shape for w in ws], [(LAT, F32)])
    return call(n, a, *ws)


def _fold_norm(mlp, nrm):
    std = jnp.maximum(nrm["std"], 1e-8)
    w1 = mlp["w"][0] / std[:, None]
    b1 = mlp["b"][0] - (nrm["mean"] / std) @ mlp["w"][0]
    out = dict(mlp)
    out["w"] = [w1] + list(mlp["w"][1:])
    out["b"] = [b1] + list(mlp["b"][1:])
    return out


def kernel(mesh_x, obj_x, mo_edge_attr, om_edge_attr, mo_edge_index,
           om_edge_index, params):
    P = params
    NMr = mesh_x.shape[0]
    NOr = obj_x.shape[0]

    m = _enc(_fold_norm(P["enc_mesh"], P["node_norm"]), mesh_x)
    o = _enc(_fold_norm(P["enc_obj"], P["node_norm"]), obj_x)
    e_mo = _enc(_fold_norm(P["enc_mo"], P["mo_norm"]), mo_edge_attr)
    e_om = _enc(_fold_norm(P["enc_om"], P["om_norm"]), om_edge_attr)

    mo_src, mo_dst = mo_edge_index[0], mo_edge_index[1]
    om_src, om_dst = om_edge_index[0], om_edge_index[1]

    for s in range(len(P["proc"])):
        ps = P["proc"][s]
        gs_mo, gd_mo, gs_om, gd_om = _sc_gather4(m, o, mo_src, mo_dst,
                                                 om_src, om_dst)
        ne_mo, e_mo = _edge_mlp(ps["e_mo"], e_mo, gs_mo, gd_mo)
        ne_om, e_om = _edge_mlp(ps["e_om"], e_om, gs_om, gd_om)
        agg_m = jax.ops.segment_sum(ne_om, om_dst, num_segments=NMr)
        agg_o = jax.ops.segment_sum(ne_mo, mo_dst, num_segments=NOr)
        m = _node_mlp(ps["n_mesh"], m, agg_m)
        o = _node_mlp(ps["n_obj"], o, agg_o)

    m_pred = _dec(P["dec_mesh"], m, 3)
    o_pred = _dec(P["dec_obj"], o, 3)
    return m_pred, o_pred
